# baseline (speedup 1.0000x reference)
import functools

import jax
import jax.numpy as jnp
from jax import lax
from jax.experimental import pallas as pl
from jax.experimental.pallas import tpu as pltpu
from jax.experimental.pallas import tpu_sc as plsc

N = 10000
E = 160000
G = 128
NP = 10240
NC = 2
NS = 16
L = 16
BR = 1000

ET = E // NS
KC = 128
GB = 10
NG = 8
EP = NS * NG * GB * KC

_MESH = dict(core_axis_name="c", subcore_axis_name="s",
             num_cores=NC, num_subcores=NS)


def _mesh():
    return plsc.VectorSubcoreMesh(**_MESH)


def _sc_params():
    return pltpu.CompilerParams(needs_layout_passes=False)


@functools.partial(
    pl.kernel,
    out_type=jax.ShapeDtypeStruct((NP // L, L), jnp.float32),
    mesh=_mesh(),
    scratch_types=[
        pltpu.VMEM((ET,), jnp.int32),
        pltpu.VMEM((NP,), jnp.float32),
        pltpu.VMEM((NS, NP // NS), jnp.float32),
        pltpu.VMEM((40, L), jnp.float32),
        pltpu.VMEM_SHARED((NS, NP), jnp.float32),
    ],
    compiler_params=_sc_params(),
)
def _deg_dinv_kernel(src_hbm, dinv_hbm, idx_v, acc_v, slab_v, dv_v, sacc):
    c = lax.axis_index("c")
    s = lax.axis_index("s")

    @pl.when(c == 0)
    def _():
        zv = jnp.zeros((L,), jnp.float32)

        def _zero(i, carry):
            acc_v[pl.ds(i * L, L)] = zv
            return carry
        lax.fori_loop(0, NP // L, _zero, 0)

        pltpu.sync_copy(src_hbm.at[s], idx_v)
        ones = jnp.full((L,), 1.0, jnp.float32)

        def _count(i, carry):
            idx = idx_v[pl.ds(i * L, L)]
            plsc.addupdate_scatter(acc_v, [idx], ones)
            return carry
        lax.fori_loop(0, ET // L, _count, 0)

        pltpu.sync_copy(acc_v, sacc.at[s])
        plsc.subcore_barrier()
        pltpu.sync_copy(sacc.at[:, pl.ds(s * (NP // NS), NP // NS)], slab_v)
        for j in range(40):
            v = slab_v[0, pl.ds(j * L, L)]
            for r in range(1, NS):
                v = v + slab_v[r, pl.ds(j * L, L)]
            i0 = plsc.bitcast(v, jnp.int32)
            y = plsc.bitcast(jnp.int32(0x5F3759DF) - (i0 >> 1), jnp.float32)
            for _ in range(3):
                y = y * (1.5 - 0.5 * v * y * y)
            dv_v[j] = jnp.where(v > 0.5, y, 0.0)
        pltpu.sync_copy(dv_v, dinv_hbm.at[pl.ds(s * 40, 40)])


def _make_p_kernel(C2):
    OR = NP // NS

    @functools.partial(
        pl.kernel,
        out_type=(jax.ShapeDtypeStruct((NP, C2), jnp.float32),
                  jax.ShapeDtypeStruct((NP, C2), jnp.float32)),
        mesh=_mesh(),
        scratch_types=[
            pltpu.VMEM((GB, KC), jnp.int32),
            pltpu.VMEM((GB, KC), jnp.int32),
            pltpu.VMEM((2, KC, C2), jnp.float32),
            pltpu.VMEM_SHARED((NP, C2), jnp.float32),
            pltpu.SemaphoreType.DMA,
            pltpu.SemaphoreType.DMA,
        ],
        compiler_params=_sc_params(),
        cost_estimate=pl.CostEstimate(
            flops=0, transcendentals=0, bytes_accessed=360_000_000),
    )
    def _p(v0, v1, src4, dst4, y0, y1, sidx, didx, rbuf, sacc, sem_g, sem_s):
        c = lax.axis_index("c")
        s = lax.axis_index("s")
        cl = C2 // L

        def body(v, y):
            zv = jnp.zeros((L,), jnp.float32)

            def _zr(i, carry):
                rbuf[0, i // cl, pl.ds((i % cl) * L, L)] = zv
                return carry
            lax.fori_loop(0, KC * cl, _zr, 0)
            for k in range(OR // KC):
                pltpu.sync_copy(rbuf.at[0],
                                sacc.at[pl.ds(s * OR + k * KC, KC)])
            plsc.subcore_barrier()

            for g in range(NG):
                pltpu.sync_copy(src4.at[s, g], sidx)
                pltpu.sync_copy(dst4.at[s, g], didx)
                pltpu.async_copy(v.at[sidx.at[0]], rbuf.at[0], sem_g)

                def chunk(k, carry):
                    b = k % 2
                    pltpu.make_async_copy(
                        v.at[sidx.at[k]], rbuf.at[b], sem_g).wait()

                    @pl.when(k >= 1)
                    def _():
                        pltpu.make_async_copy(
                            rbuf.at[1 - b],
                            sacc.at[didx.at[k - 1]], sem_s).wait()

                    @pl.when(k + 1 < GB)
                    def _():
                        pltpu.async_copy(v.at[sidx.at[k + 1]],
                                         rbuf.at[1 - b], sem_g)
                    pltpu.async_copy(rbuf.at[b], sacc.at[didx.at[k]],
                                     sem_s, add=True)
                    return carry
                lax.fori_loop(0, GB, chunk, 0)
                pltpu.make_async_copy(
                    rbuf.at[(GB - 1) % 2],
                    sacc.at[didx.at[GB - 1]], sem_s).wait()
            plsc.subcore_barrier()

            pltpu.sync_copy(sacc.at[pl.ds(s * OR, OR)], y.at[pl.ds(s * OR, OR)])

        @pl.when(c == 0)
        def _():
            body(v0, y0)

        @pl.when(c == 1)
        def _():
            body(v1, y1)

    return _p


PC = 128
_p_kernel_inst = []


def _p_apply(va, vb, src3, dst3):
    if not _p_kernel_inst:
        _p_kernel_inst.append(_make_p_kernel(PC))
    return _p_kernel_inst[0](va, vb, src3, dst3)


def _make_p2_kernel():
    C2 = PC
    OR = NP // NS
    NG2 = NG // NC

    @functools.partial(
        pl.kernel,
        out_type=(jax.ShapeDtypeStruct((NP, C2), jnp.float32),
                  jax.ShapeDtypeStruct((NP, C2), jnp.float32)),
        mesh=_mesh(),
        scratch_types=[
            pltpu.VMEM((GB, KC), jnp.int32),
            pltpu.VMEM((GB, KC), jnp.int32),
            pltpu.VMEM((2, KC, C2), jnp.float32),
            pltpu.VMEM_SHARED((NP, C2), jnp.float32),
            pltpu.SemaphoreType.DMA,
            pltpu.SemaphoreType.DMA,
        ],
        compiler_params=_sc_params(),
        cost_estimate=pl.CostEstimate(
            flops=0, transcendentals=0, bytes_accessed=200_000_000),
    )
    def _p2(v, src5, dst5, y0, y1, sidx, didx, rbuf, sacc, sem_g, sem_s):
        c = lax.axis_index("c")
        s = lax.axis_index("s")
        cl = C2 // L

        def body(y):
            zv = jnp.zeros((L,), jnp.float32)

            def _zr(i, carry):
                rbuf[0, i // cl, pl.ds((i % cl) * L, L)] = zv
                return carry
            lax.fori_loop(0, KC * cl, _zr, 0)
            for k in range(OR // KC):
                pltpu.sync_copy(rbuf.at[0],
                                sacc.at[pl.ds(s * OR + k * KC, KC)])
            plsc.subcore_barrier()

            for g in range(NG2):
                pltpu.sync_copy(src5.at[c, s, g], sidx)
                pltpu.sync_copy(dst5.at[c, s, g], didx)
                pltpu.async_copy(v.at[sidx.at[0]], rbuf.at[0], sem_g)

                def chunk(k, carry):
                    b = k % 2
                    pltpu.make_async_copy(
                        v.at[sidx.at[k]], rbuf.at[b], sem_g).wait()

                    @pl.when(k >= 1)
                    def _():
                        pltpu.make_async_copy(
                            rbuf.at[1 - b],
                            sacc.at[didx.at[k - 1]], sem_s).wait()

                    @pl.when(k + 1 < GB)
                    def _():
                        pltpu.async_copy(v.at[sidx.at[k + 1]],
                                         rbuf.at[1 - b], sem_g)
                    pltpu.async_copy(rbuf.at[b], sacc.at[didx.at[k]],
                                     sem_s, add=True)
                    return carry
                lax.fori_loop(0, GB, chunk, 0)
                pltpu.make_async_copy(
                    rbuf.at[(GB - 1) % 2],
                    sacc.at[didx.at[GB - 1]], sem_s).wait()
            plsc.subcore_barrier()
            pltpu.sync_copy(sacc.at[pl.ds(s * OR, OR)], y.at[pl.ds(s * OR, OR)])

        @pl.when(c == 0)
        def _():
            body(y0)

        @pl.when(c == 1)
        def _():
            body(y1)

    return _p2


_p2_kernel_inst = []


def _p2_apply(v, src5, dst5):
    if not _p2_kernel_inst:
        _p2_kernel_inst.append(_make_p2_kernel())
    return _p2_kernel_inst[0](v, src5, dst5)


def _lead(x, dinv_col, W, b):
    C = x.shape[1]
    C2 = C // 2
    Co = W.shape[2]

    def body(x_ref, d_ref, w_ref, b_ref, ua_ref, ub_ref, acc_ref):
        xv = x_ref[...]
        u = xv * d_ref[...]
        ua_ref[...] = u[:, :C2]
        ub_ref[...] = u[:, C2:]
        w0 = w_ref[0] - w_ref[2]
        acc_ref[...] = (jnp.dot(xv, w0, preferred_element_type=jnp.float32)
                        + b_ref[...])

    return pl.pallas_call(
        body,
        grid=(N // BR,),
        in_specs=[
            pl.BlockSpec((BR, C), lambda i: (i, 0)),
            pl.BlockSpec((BR, 1), lambda i: (i, 0)),
            pl.BlockSpec((3, C, Co), lambda i: (0, 0, 0)),
            pl.BlockSpec((1, Co), lambda i: (0, 0)),
        ],
        out_specs=[
            pl.BlockSpec((BR, C2), lambda i: (i, 0)),
            pl.BlockSpec((BR, C2), lambda i: (i, 0)),
            pl.BlockSpec((BR, Co), lambda i: (i, 0)),
        ],
        out_shape=[
            jax.ShapeDtypeStruct((N, C2), jnp.float32),
            jax.ShapeDtypeStruct((N, C2), jnp.float32),
            jax.ShapeDtypeStruct((N, Co), jnp.float32),
        ],
    )(x, dinv_col, W, b)


def _scale1(sa, sb, dinv_col, C2):
    def body(sa_ref, sb_ref, d_ref, ua_ref, ub_ref):
        d = d_ref[...]
        f = -(d * d)
        ua_ref[...] = sa_ref[...] * f
        ub_ref[...] = sb_ref[...] * f

    return pl.pallas_call(
        body,
        grid=(N // BR,),
        in_specs=[
            pl.BlockSpec((BR, PC), lambda i: (i, 0)),
            pl.BlockSpec((BR, PC), lambda i: (i, 0)),
            pl.BlockSpec((BR, 1), lambda i: (i, 0)),
        ],
        out_specs=[
            pl.BlockSpec((BR, PC), lambda i: (i, 0)),
            pl.BlockSpec((BR, PC), lambda i: (i, 0)),
        ],
        out_shape=[
            jax.ShapeDtypeStruct((N, PC), jnp.float32),
            jax.ShapeDtypeStruct((N, PC), jnp.float32),
        ],
    )(sa, sb, dinv_col)


def _scale1_sum(sa, sb, dinv_col):
    def body(sa_ref, sb_ref, d_ref, u_ref):
        d = d_ref[...]
        u_ref[...] = (sa_ref[...] + sb_ref[...]) * (-(d * d))

    return pl.pallas_call(
        body,
        grid=(N // BR,),
        in_specs=[
            pl.BlockSpec((BR, PC), lambda i: (i, 0)),
            pl.BlockSpec((BR, PC), lambda i: (i, 0)),
            pl.BlockSpec((BR, 1), lambda i: (i, 0)),
        ],
        out_specs=pl.BlockSpec((BR, PC), lambda i: (i, 0)),
        out_shape=jax.ShapeDtypeStruct((N, PC), jnp.float32),
    )(sa, sb, dinv_col)


def _post_pre(acc0, s1a, s1b, s2a, s2b, dinv_col, W, Wn, bn, s_mode, u_out):
    Ci = W.shape[1]
    Co = W.shape[2]
    Con = Wn.shape[2]
    C2o = Co // 2

    def body(a_ref, s1a_ref, s1b_ref, s2a_ref, s2b_ref, d_ref, w_ref,
             wn_ref, bn_ref, an_ref, *u_refs):
        d = d_ref[...]
        if s_mode == "halves":
            s1 = jnp.concatenate([s1a_ref[...], s1b_ref[...]], axis=1) * d
            s2 = jnp.concatenate([s2a_ref[...], s2b_ref[...]], axis=1) * d
        else:
            s1 = (s1a_ref[...] + s1b_ref[...]) * d
            s2 = (s2a_ref[...] + s2b_ref[...]) * d
        acc = a_ref[...] - jnp.dot(s1, w_ref[1],
                                   preferred_element_type=jnp.float32)
        acc -= 2.0 * jnp.dot(s2, w_ref[2], preferred_element_type=jnp.float32)
        h_out = jnp.maximum(acc, 0.0)
        wn0 = wn_ref[0] - wn_ref[2]
        an_ref[...] = (jnp.dot(h_out, wn0, preferred_element_type=jnp.float32)
                       + bn_ref[...])
        if u_out == "full":
            u_refs[0][...] = h_out * d
        elif u_out == "halves":
            u = h_out * d
            u_refs[0][...] = u[:, :C2o]
            u_refs[1][...] = u[:, C2o:]

    out_shape = [jax.ShapeDtypeStruct((N, Con), jnp.float32)]
    out_specs = [pl.BlockSpec((BR, Con), lambda i: (i, 0))]
    if u_out == "full":
        out_shape += [jax.ShapeDtypeStruct((N, Co), jnp.float32)]
        out_specs += [pl.BlockSpec((BR, Co), lambda i: (i, 0))]
    elif u_out == "halves":
        out_shape += [jax.ShapeDtypeStruct((N, C2o), jnp.float32)] * 2
        out_specs += [pl.BlockSpec((BR, C2o), lambda i: (i, 0))] * 2

    return pl.pallas_call(
        body,
        grid=(N // BR,),
        in_specs=[
            pl.BlockSpec((BR, Co), lambda i: (i, 0)),
            pl.BlockSpec((BR, PC), lambda i: (i, 0)),
            pl.BlockSpec((BR, PC), lambda i: (i, 0)),
            pl.BlockSpec((BR, PC), lambda i: (i, 0)),
            pl.BlockSpec((BR, PC), lambda i: (i, 0)),
            pl.BlockSpec((BR, 1), lambda i: (i, 0)),
            pl.BlockSpec((3, Ci, Co), lambda i: (0, 0, 0)),
            pl.BlockSpec((3, Co, Con), lambda i: (0, 0, 0)),
            pl.BlockSpec((1, Con), lambda i: (0, 0)),
        ],
        out_specs=out_specs,
        out_shape=out_shape,
    )(acc0, s1a, s1b, s2a, s2b, dinv_col, W, Wn, bn)


def _cheb_post_pool(acc0, s1a, s1b, s2a, s2b, dinv_col, W, batch3, Wfc,
                    bfc_row):
    Ci = W.shape[1]
    Co = W.shape[2]
    NB = N // BR

    def body(a_ref, s1a_ref, s1b_ref, s2a_ref, s2b_ref, d_ref, w_ref,
             batch_ref, wfc_ref, bfc_ref, out_ref, sums, cnt):
        i = pl.program_id(0)

        @pl.when(i == 0)
        def _():
            sums[...] = jnp.zeros_like(sums)
            cnt[...] = jnp.zeros_like(cnt)

        d = d_ref[...]
        s1 = jnp.concatenate([s1a_ref[...], s1b_ref[...]], axis=1) * d
        s2 = jnp.concatenate([s2a_ref[...], s2b_ref[...]], axis=1) * d
        acc = a_ref[...] - jnp.dot(s1, w_ref[1],
                                   preferred_element_type=jnp.float32)
        acc -= 2.0 * jnp.dot(s2, w_ref[2], preferred_element_type=jnp.float32)
        h3 = jnp.maximum(acc, 0.0)

        mt = (lax.broadcasted_iota(jnp.int32, (G, BR), 0)
              == batch_ref[...][0]).astype(jnp.float32)
        sums[...] += jnp.dot(mt, h3, preferred_element_type=jnp.float32)
        cnt[...] += jnp.sum(mt, axis=1, keepdims=True)

        @pl.when(i == NB - 1)
        def _():
            pooled = sums[...] / jnp.maximum(cnt[...], 1.0)
            out_ref[...] = (jnp.dot(pooled, wfc_ref[...],
                                    preferred_element_type=jnp.float32)
                            + bfc_ref[...])

    return pl.pallas_call(
        body,
        grid=(NB,),
        in_specs=[
            pl.BlockSpec((BR, Co), lambda i: (i, 0)),
            pl.BlockSpec((BR, PC), lambda i: (i, 0)),
            pl.BlockSpec((BR, PC), lambda i: (i, 0)),
            pl.BlockSpec((BR, PC), lambda i: (i, 0)),
            pl.BlockSpec((BR, PC), lambda i: (i, 0)),
            pl.BlockSpec((BR, 1), lambda i: (i, 0)),
            pl.BlockSpec((3, Ci, Co), lambda i: (0, 0, 0)),
            pl.BlockSpec((1, 1, BR), lambda i: (i, 0, 0)),
            pl.BlockSpec(Wfc.shape, lambda i: (0, 0)),
            pl.BlockSpec((1, 128), lambda i: (0, 0)),
        ],
        out_specs=pl.BlockSpec((G, 128), lambda i: (0, 0)),
        out_shape=jax.ShapeDtypeStruct((G, 128), jnp.float32),
        scratch_shapes=[
            pltpu.VMEM((G, Co), jnp.float32),
            pltpu.VMEM((G, 1), jnp.float32),
        ],
    )(acc0, s1a, s1b, s2a, s2b, dinv_col, W, batch3, Wfc, bfc_row)


def kernel(x, edge_index, batch, W1, b1, W2, b2, W3, b3, Wfc, bfc):
    src = edge_index[0]
    dst = edge_index[1]
    src_deg = src.reshape(NS, ET)
    pad = EP - E
    src_p = jnp.concatenate(
        [src, (jnp.arange(pad, dtype=jnp.int32) * 13) % N])
    dst_p = jnp.concatenate([dst, jnp.full((pad,), NP - 1, jnp.int32)])
    src3 = src_p.reshape(NS, NG, GB, KC)
    dst3 = dst_p.reshape(NS, NG, GB, KC)
    src5 = src_p.reshape(NC, NS, NG // NC, GB, KC)
    dst5 = dst_p.reshape(NC, NS, NG // NC, GB, KC)

    dinv2d = _deg_dinv_kernel(src_deg)
    dinv_col = dinv2d.reshape(NP)[:N].reshape(N, 1)

    ua, ub, pre1 = _lead(x, dinv_col, W1, b1.reshape(1, -1))
    s1a, s1b = _p_apply(ua, ub, src3, dst3)
    u1a, u1b = _scale1(s1a, s1b, dinv_col, PC)
    s2a, s2b = _p_apply(u1a, u1b, src3, dst3)
    pre2, uf = _post_pre(pre1, s1a, s1b, s2a, s2b, dinv_col, W1, W2,
                         b2.reshape(1, -1), s_mode="halves", u_out="full")

    t1a, t1b = _p2_apply(uf, src5, dst5)
    u2f = _scale1_sum(t1a, t1b, dinv_col)
    t2a, t2b = _p2_apply(u2f, src5, dst5)
    pre3, ua, ub = _post_pre(pre2, t1a, t1b, t2a, t2b, dinv_col, W2, W3,
                             b3.reshape(1, -1), s_mode="partials",
                             u_out="halves")

    s1a, s1b = _p_apply(ua, ub, src3, dst3)
    u1a, u1b = _scale1(s1a, s1b, dinv_col, PC)
    s2a, s2b = _p_apply(u1a, u1b, src3, dst3)
    return _cheb_post_pool(pre3, s1a, s1b, s2a, s2b, dinv_col, W3,
                           batch.reshape(N // BR, 1, BR), Wfc,
                           bfc.reshape(1, -1))


# speedup vs baseline: 1.1958x; 1.1958x over previous
import functools

import jax
import jax.numpy as jnp
from jax import lax
from jax.experimental import pallas as pl
from jax.experimental.pallas import tpu as pltpu
from jax.experimental.pallas import tpu_sc as plsc

N = 10000
E = 160000
G = 128
NP = 10240
NC = 2
NS = 16
L = 16
BR = 1000

ET = E // NS
KC = 80
GB = 16
NG = 8
NB3 = 3
EP = NS * NG * GB * KC

_MESH = dict(core_axis_name="c", subcore_axis_name="s",
             num_cores=NC, num_subcores=NS)


def _mesh():
    return plsc.VectorSubcoreMesh(**_MESH)


def _sc_params():
    return pltpu.CompilerParams(needs_layout_passes=False)


@functools.partial(
    pl.kernel,
    out_type=jax.ShapeDtypeStruct((NP // L, L), jnp.float32),
    mesh=_mesh(),
    scratch_types=[
        pltpu.VMEM((ET,), jnp.int32),
        pltpu.VMEM((NP,), jnp.float32),
        pltpu.VMEM((NS, NP // NS), jnp.float32),
        pltpu.VMEM((40, L), jnp.float32),
        pltpu.VMEM_SHARED((NS, NP), jnp.float32),
    ],
    compiler_params=_sc_params(),
)
def _deg_dinv_kernel(src_hbm, dinv_hbm, idx_v, acc_v, slab_v, dv_v, sacc):
    c = lax.axis_index("c")
    s = lax.axis_index("s")

    @pl.when(c == 0)
    def _():
        zv = jnp.zeros((L,), jnp.float32)

        def _zero(i, carry):
            acc_v[pl.ds(i * L, L)] = zv
            return carry
        lax.fori_loop(0, NP // L, _zero, 0)

        pltpu.sync_copy(src_hbm.at[s], idx_v)
        ones = jnp.full((L,), 1.0, jnp.float32)

        def _count(i, carry):
            idx = idx_v[pl.ds(i * L, L)]
            plsc.addupdate_scatter(acc_v, [idx], ones)
            return carry
        lax.fori_loop(0, ET // L, _count, 0)

        pltpu.sync_copy(acc_v, sacc.at[s])
        plsc.subcore_barrier()
        pltpu.sync_copy(sacc.at[:, pl.ds(s * (NP // NS), NP // NS)], slab_v)
        for j in range(40):
            v = slab_v[0, pl.ds(j * L, L)]
            for r in range(1, NS):
                v = v + slab_v[r, pl.ds(j * L, L)]
            i0 = plsc.bitcast(v, jnp.int32)
            y = plsc.bitcast(jnp.int32(0x5F3759DF) - (i0 >> 1), jnp.float32)
            for _ in range(3):
                y = y * (1.5 - 0.5 * v * y * y)
            dv_v[j] = jnp.where(v > 0.5, y, 0.0)
        pltpu.sync_copy(dv_v, dinv_hbm.at[pl.ds(s * 40, 40)])


def _make_p_kernel(C2):
    OR = NP // NS

    @functools.partial(
        pl.kernel,
        out_type=(jax.ShapeDtypeStruct((NP, C2), jnp.float32),
                  jax.ShapeDtypeStruct((NP, C2), jnp.float32)),
        mesh=_mesh(),
        scratch_types=[
            pltpu.VMEM((GB, KC), jnp.int32),
            pltpu.VMEM((GB, KC), jnp.int32),
            pltpu.VMEM((NB3, KC, C2), jnp.float32),
            pltpu.VMEM_SHARED((NP, C2), jnp.float32),
            pltpu.SemaphoreType.DMA,
            pltpu.SemaphoreType.DMA,
        ],
        compiler_params=_sc_params(),
        cost_estimate=pl.CostEstimate(
            flops=0, transcendentals=0, bytes_accessed=360_000_000),
    )
    def _p(v0, v1, src4, dst4, y0, y1, sidx, didx, rbuf, sacc, sem_g, sem_s):
        c = lax.axis_index("c")
        s = lax.axis_index("s")
        cl = C2 // L

        def body(v, y):
            zv = jnp.zeros((L,), jnp.float32)

            def _zr(i, carry):
                rbuf[0, i // cl, pl.ds((i % cl) * L, L)] = zv
                return carry
            lax.fori_loop(0, KC * cl, _zr, 0)
            for k in range(OR // KC):
                pltpu.sync_copy(rbuf.at[0],
                                sacc.at[pl.ds(s * OR + k * KC, KC)])
            plsc.subcore_barrier()

            for g in range(NG):
                pltpu.sync_copy(src4.at[s, g], sidx)
                pltpu.sync_copy(dst4.at[s, g], didx)
                pltpu.async_copy(v.at[sidx.at[0]], rbuf.at[0], sem_g)
                pltpu.async_copy(v.at[sidx.at[1]], rbuf.at[1], sem_g)

                def chunk(k, carry):
                    b = k % NB3
                    pltpu.make_async_copy(
                        v.at[sidx.at[k]], rbuf.at[b], sem_g).wait()

                    @pl.when(k >= 1)
                    def _():
                        pltpu.make_async_copy(
                            rbuf.at[(k - 1) % NB3],
                            sacc.at[didx.at[k - 1]], sem_s).wait()

                    @pl.when(k + 2 < GB)
                    def _():
                        pltpu.async_copy(v.at[sidx.at[k + 2]],
                                         rbuf.at[(k + 2) % NB3], sem_g)
                    pltpu.async_copy(rbuf.at[b], sacc.at[didx.at[k]],
                                     sem_s, add=True)
                    return carry
                lax.fori_loop(0, GB, chunk, 0)
                pltpu.make_async_copy(
                    rbuf.at[(GB - 1) % NB3],
                    sacc.at[didx.at[GB - 1]], sem_s).wait()
            plsc.subcore_barrier()

            pltpu.sync_copy(sacc.at[pl.ds(s * OR, OR)], y.at[pl.ds(s * OR, OR)])

        @pl.when(c == 0)
        def _():
            body(v0, y0)

        @pl.when(c == 1)
        def _():
            body(v1, y1)

    return _p


PC = 128
_p_kernel_inst = []


def _p_apply(va, vb, src3, dst3):
    if not _p_kernel_inst:
        _p_kernel_inst.append(_make_p_kernel(PC))
    return _p_kernel_inst[0](va, vb, src3, dst3)


def _make_p2_kernel():
    C2 = PC
    OR = NP // NS
    NG2 = NG // NC

    @functools.partial(
        pl.kernel,
        out_type=(jax.ShapeDtypeStruct((NP, C2), jnp.float32),
                  jax.ShapeDtypeStruct((NP, C2), jnp.float32)),
        mesh=_mesh(),
        scratch_types=[
            pltpu.VMEM((GB, KC), jnp.int32),
            pltpu.VMEM((GB, KC), jnp.int32),
            pltpu.VMEM((NB3, KC, C2), jnp.float32),
            pltpu.VMEM_SHARED((NP, C2), jnp.float32),
            pltpu.SemaphoreType.DMA,
            pltpu.SemaphoreType.DMA,
        ],
        compiler_params=_sc_params(),
        cost_estimate=pl.CostEstimate(
            flops=0, transcendentals=0, bytes_accessed=200_000_000),
    )
    def _p2(v, src5, dst5, y0, y1, sidx, didx, rbuf, sacc, sem_g, sem_s):
        c = lax.axis_index("c")
        s = lax.axis_index("s")
        cl = C2 // L

        def body(y):
            zv = jnp.zeros((L,), jnp.float32)

            def _zr(i, carry):
                rbuf[0, i // cl, pl.ds((i % cl) * L, L)] = zv
                return carry
            lax.fori_loop(0, KC * cl, _zr, 0)
            for k in range(OR // KC):
                pltpu.sync_copy(rbuf.at[0],
                                sacc.at[pl.ds(s * OR + k * KC, KC)])
            plsc.subcore_barrier()

            for g in range(NG2):
                pltpu.sync_copy(src5.at[c, s, g], sidx)
                pltpu.sync_copy(dst5.at[c, s, g], didx)
                pltpu.async_copy(v.at[sidx.at[0]], rbuf.at[0], sem_g)
                pltpu.async_copy(v.at[sidx.at[1]], rbuf.at[1], sem_g)

                def chunk(k, carry):
                    b = k % NB3
                    pltpu.make_async_copy(
                        v.at[sidx.at[k]], rbuf.at[b], sem_g).wait()

                    @pl.when(k >= 1)
                    def _():
                        pltpu.make_async_copy(
                            rbuf.at[(k - 1) % NB3],
                            sacc.at[didx.at[k - 1]], sem_s).wait()

                    @pl.when(k + 2 < GB)
                    def _():
                        pltpu.async_copy(v.at[sidx.at[k + 2]],
                                         rbuf.at[(k + 2) % NB3], sem_g)
                    pltpu.async_copy(rbuf.at[b], sacc.at[didx.at[k]],
                                     sem_s, add=True)
                    return carry
                lax.fori_loop(0, GB, chunk, 0)
                pltpu.make_async_copy(
                    rbuf.at[(GB - 1) % NB3],
                    sacc.at[didx.at[GB - 1]], sem_s).wait()
            plsc.subcore_barrier()
            pltpu.sync_copy(sacc.at[pl.ds(s * OR, OR)], y.at[pl.ds(s * OR, OR)])

        @pl.when(c == 0)
        def _():
            body(y0)

        @pl.when(c == 1)
        def _():
            body(y1)

    return _p2


_p2_kernel_inst = []


def _p2_apply(v, src5, dst5):
    if not _p2_kernel_inst:
        _p2_kernel_inst.append(_make_p2_kernel())
    return _p2_kernel_inst[0](v, src5, dst5)


def _lead(x, dinv_col, W, b):
    C = x.shape[1]
    C2 = C // 2
    Co = W.shape[2]

    def body(x_ref, d_ref, w_ref, b_ref, ua_ref, ub_ref, acc_ref):
        xv = x_ref[...]
        u = xv * d_ref[...]
        ua_ref[...] = u[:, :C2]
        ub_ref[...] = u[:, C2:]
        w0 = w_ref[0] - w_ref[2]
        acc_ref[...] = (jnp.dot(xv, w0, preferred_element_type=jnp.float32)
                        + b_ref[...])

    return pl.pallas_call(
        body,
        grid=(N // BR,),
        in_specs=[
            pl.BlockSpec((BR, C), lambda i: (i, 0)),
            pl.BlockSpec((BR, 1), lambda i: (i, 0)),
            pl.BlockSpec((3, C, Co), lambda i: (0, 0, 0)),
            pl.BlockSpec((1, Co), lambda i: (0, 0)),
        ],
        out_specs=[
            pl.BlockSpec((BR, C2), lambda i: (i, 0)),
            pl.BlockSpec((BR, C2), lambda i: (i, 0)),
            pl.BlockSpec((BR, Co), lambda i: (i, 0)),
        ],
        out_shape=[
            jax.ShapeDtypeStruct((N, C2), jnp.float32),
            jax.ShapeDtypeStruct((N, C2), jnp.float32),
            jax.ShapeDtypeStruct((N, Co), jnp.float32),
        ],
    )(x, dinv_col, W, b)


def _scale1(sa, sb, dinv_col, C2):
    def body(sa_ref, sb_ref, d_ref, ua_ref, ub_ref):
        d = d_ref[...]
        f = -(d * d)
        ua_ref[...] = sa_ref[...] * f
        ub_ref[...] = sb_ref[...] * f

    return pl.pallas_call(
        body,
        grid=(N // BR,),
        in_specs=[
            pl.BlockSpec((BR, PC), lambda i: (i, 0)),
            pl.BlockSpec((BR, PC), lambda i: (i, 0)),
            pl.BlockSpec((BR, 1), lambda i: (i, 0)),
        ],
        out_specs=[
            pl.BlockSpec((BR, PC), lambda i: (i, 0)),
            pl.BlockSpec((BR, PC), lambda i: (i, 0)),
        ],
        out_shape=[
            jax.ShapeDtypeStruct((N, PC), jnp.float32),
            jax.ShapeDtypeStruct((N, PC), jnp.float32),
        ],
    )(sa, sb, dinv_col)


def _scale1_sum(sa, sb, dinv_col):
    def body(sa_ref, sb_ref, d_ref, u_ref):
        d = d_ref[...]
        u_ref[...] = (sa_ref[...] + sb_ref[...]) * (-(d * d))

    return pl.pallas_call(
        body,
        grid=(N // BR,),
        in_specs=[
            pl.BlockSpec((BR, PC), lambda i: (i, 0)),
            pl.BlockSpec((BR, PC), lambda i: (i, 0)),
            pl.BlockSpec((BR, 1), lambda i: (i, 0)),
        ],
        out_specs=pl.BlockSpec((BR, PC), lambda i: (i, 0)),
        out_shape=jax.ShapeDtypeStruct((N, PC), jnp.float32),
    )(sa, sb, dinv_col)


def _post_pre(acc0, s1a, s1b, s2a, s2b, dinv_col, W, Wn, bn, s_mode, u_out):
    Ci = W.shape[1]
    Co = W.shape[2]
    Con = Wn.shape[2]
    C2o = Co // 2

    def body(a_ref, s1a_ref, s1b_ref, s2a_ref, s2b_ref, d_ref, w_ref,
             wn_ref, bn_ref, an_ref, *u_refs):
        d = d_ref[...]
        if s_mode == "halves":
            s1 = jnp.concatenate([s1a_ref[...], s1b_ref[...]], axis=1) * d
            s2 = jnp.concatenate([s2a_ref[...], s2b_ref[...]], axis=1) * d
        else:
            s1 = (s1a_ref[...] + s1b_ref[...]) * d
            s2 = (s2a_ref[...] + s2b_ref[...]) * d
        acc = a_ref[...] - jnp.dot(s1, w_ref[1],
                                   preferred_element_type=jnp.float32)
        acc -= 2.0 * jnp.dot(s2, w_ref[2], preferred_element_type=jnp.float32)
        h_out = jnp.maximum(acc, 0.0)
        wn0 = wn_ref[0] - wn_ref[2]
        an_ref[...] = (jnp.dot(h_out, wn0, preferred_element_type=jnp.float32)
                       + bn_ref[...])
        if u_out == "full":
            u_refs[0][...] = h_out * d
        elif u_out == "halves":
            u = h_out * d
            u_refs[0][...] = u[:, :C2o]
            u_refs[1][...] = u[:, C2o:]

    out_shape = [jax.ShapeDtypeStruct((N, Con), jnp.float32)]
    out_specs = [pl.BlockSpec((BR, Con), lambda i: (i, 0))]
    if u_out == "full":
        out_shape += [jax.ShapeDtypeStruct((N, Co), jnp.float32)]
        out_specs += [pl.BlockSpec((BR, Co), lambda i: (i, 0))]
    elif u_out == "halves":
        out_shape += [jax.ShapeDtypeStruct((N, C2o), jnp.float32)] * 2
        out_specs += [pl.BlockSpec((BR, C2o), lambda i: (i, 0))] * 2

    return pl.pallas_call(
        body,
        grid=(N // BR,),
        in_specs=[
            pl.BlockSpec((BR, Co), lambda i: (i, 0)),
            pl.BlockSpec((BR, PC), lambda i: (i, 0)),
            pl.BlockSpec((BR, PC), lambda i: (i, 0)),
            pl.BlockSpec((BR, PC), lambda i: (i, 0)),
            pl.BlockSpec((BR, PC), lambda i: (i, 0)),
            pl.BlockSpec((BR, 1), lambda i: (i, 0)),
            pl.BlockSpec((3, Ci, Co), lambda i: (0, 0, 0)),
            pl.BlockSpec((3, Co, Con), lambda i: (0, 0, 0)),
            pl.BlockSpec((1, Con), lambda i: (0, 0)),
        ],
        out_specs=out_specs,
        out_shape=out_shape,
    )(acc0, s1a, s1b, s2a, s2b, dinv_col, W, Wn, bn)


def _cheb_post_pool(acc0, s1a, s1b, s2a, s2b, dinv_col, W, batch3, Wfc,
                    bfc_row):
    Ci = W.shape[1]
    Co = W.shape[2]
    NB = N // BR

    def body(a_ref, s1a_ref, s1b_ref, s2a_ref, s2b_ref, d_ref, w_ref,
             batch_ref, wfc_ref, bfc_ref, out_ref, sums, cnt):
        i = pl.program_id(0)

        @pl.when(i == 0)
        def _():
            sums[...] = jnp.zeros_like(sums)
            cnt[...] = jnp.zeros_like(cnt)

        d = d_ref[...]
        s1 = jnp.concatenate([s1a_ref[...], s1b_ref[...]], axis=1) * d
        s2 = jnp.concatenate([s2a_ref[...], s2b_ref[...]], axis=1) * d
        acc = a_ref[...] - jnp.dot(s1, w_ref[1],
                                   preferred_element_type=jnp.float32)
        acc -= 2.0 * jnp.dot(s2, w_ref[2], preferred_element_type=jnp.float32)
        h3 = jnp.maximum(acc, 0.0)

        mt = (lax.broadcasted_iota(jnp.int32, (G, BR), 0)
              == batch_ref[...][0]).astype(jnp.float32)
        sums[...] += jnp.dot(mt, h3, preferred_element_type=jnp.float32)
        cnt[...] += jnp.sum(mt, axis=1, keepdims=True)

        @pl.when(i == NB - 1)
        def _():
            pooled = sums[...] / jnp.maximum(cnt[...], 1.0)
            out_ref[...] = (jnp.dot(pooled, wfc_ref[...],
                                    preferred_element_type=jnp.float32)
                            + bfc_ref[...])

    return pl.pallas_call(
        body,
        grid=(NB,),
        in_specs=[
            pl.BlockSpec((BR, Co), lambda i: (i, 0)),
            pl.BlockSpec((BR, PC), lambda i: (i, 0)),
            pl.BlockSpec((BR, PC), lambda i: (i, 0)),
            pl.BlockSpec((BR, PC), lambda i: (i, 0)),
            pl.BlockSpec((BR, PC), lambda i: (i, 0)),
            pl.BlockSpec((BR, 1), lambda i: (i, 0)),
            pl.BlockSpec((3, Ci, Co), lambda i: (0, 0, 0)),
            pl.BlockSpec((1, 1, BR), lambda i: (i, 0, 0)),
            pl.BlockSpec(Wfc.shape, lambda i: (0, 0)),
            pl.BlockSpec((1, 128), lambda i: (0, 0)),
        ],
        out_specs=pl.BlockSpec((G, 128), lambda i: (0, 0)),
        out_shape=jax.ShapeDtypeStruct((G, 128), jnp.float32),
        scratch_shapes=[
            pltpu.VMEM((G, Co), jnp.float32),
            pltpu.VMEM((G, 1), jnp.float32),
        ],
    )(acc0, s1a, s1b, s2a, s2b, dinv_col, W, batch3, Wfc, bfc_row)


def kernel(x, edge_index, batch, W1, b1, W2, b2, W3, b3, Wfc, bfc):
    src = edge_index[0]
    dst = edge_index[1]
    src_deg = src.reshape(NS, ET)
    pad = EP - E
    src_p = jnp.concatenate(
        [src, (jnp.arange(pad, dtype=jnp.int32) * 13) % N])
    dst_p = jnp.concatenate([dst, jnp.full((pad,), NP - 1, jnp.int32)])
    src3 = src_p.reshape(NS, NG, GB, KC)
    dst3 = dst_p.reshape(NS, NG, GB, KC)
    src5 = src_p.reshape(NC, NS, NG // NC, GB, KC)
    dst5 = dst_p.reshape(NC, NS, NG // NC, GB, KC)

    dinv2d = _deg_dinv_kernel(src_deg)
    dinv_col = dinv2d.reshape(NP)[:N].reshape(N, 1)

    ua, ub, pre1 = _lead(x, dinv_col, W1, b1.reshape(1, -1))
    s1a, s1b = _p_apply(ua, ub, src3, dst3)
    u1a, u1b = _scale1(s1a, s1b, dinv_col, PC)
    s2a, s2b = _p_apply(u1a, u1b, src3, dst3)
    pre2, uf = _post_pre(pre1, s1a, s1b, s2a, s2b, dinv_col, W1, W2,
                         b2.reshape(1, -1), s_mode="halves", u_out="full")

    t1a, t1b = _p2_apply(uf, src5, dst5)
    u2f = _scale1_sum(t1a, t1b, dinv_col)
    t2a, t2b = _p2_apply(u2f, src5, dst5)
    pre3, ua, ub = _post_pre(pre2, t1a, t1b, t2a, t2b, dinv_col, W2, W3,
                             b3.reshape(1, -1), s_mode="partials",
                             u_out="halves")

    s1a, s1b = _p_apply(ua, ub, src3, dst3)
    u1a, u1b = _scale1(s1a, s1b, dinv_col, PC)
    s2a, s2b = _p_apply(u1a, u1b, src3, dst3)
    return _cheb_post_pool(pre3, s1a, s1b, s2a, s2b, dinv_col, W3,
                           batch.reshape(N // BR, 1, BR), Wfc,
                           bfc.reshape(1, -1))


# speedup vs baseline: 1.2190x; 1.0195x over previous
import functools

import jax
import jax.numpy as jnp
from jax import lax
from jax.experimental import pallas as pl
from jax.experimental.pallas import tpu as pltpu
from jax.experimental.pallas import tpu_sc as plsc

N = 10000
E = 160000
G = 128
NP = 10240
NC = 2
NS = 16
L = 16
BR = 1000

ET = E // NS
KC = 64
GB = 16
NG = 10
NB3 = 4
EP = NS * NG * GB * KC

_MESH = dict(core_axis_name="c", subcore_axis_name="s",
             num_cores=NC, num_subcores=NS)


def _mesh():
    return plsc.VectorSubcoreMesh(**_MESH)


def _sc_params():
    return pltpu.CompilerParams(needs_layout_passes=False)


@functools.partial(
    pl.kernel,
    out_type=jax.ShapeDtypeStruct((NP // L, L), jnp.float32),
    mesh=_mesh(),
    scratch_types=[
        pltpu.VMEM((ET,), jnp.int32),
        pltpu.VMEM((NP,), jnp.float32),
        pltpu.VMEM((NS, NP // NS), jnp.float32),
        pltpu.VMEM((40, L), jnp.float32),
        pltpu.VMEM_SHARED((NS, NP), jnp.float32),
    ],
    compiler_params=_sc_params(),
)
def _deg_dinv_kernel(src_hbm, dinv_hbm, idx_v, acc_v, slab_v, dv_v, sacc):
    c = lax.axis_index("c")
    s = lax.axis_index("s")

    @pl.when(c == 0)
    def _():
        zv = jnp.zeros((L,), jnp.float32)

        def _zero(i, carry):
            acc_v[pl.ds(i * L, L)] = zv
            return carry
        lax.fori_loop(0, NP // L, _zero, 0)

        pltpu.sync_copy(src_hbm.at[s], idx_v)
        ones = jnp.full((L,), 1.0, jnp.float32)

        def _count(i, carry):
            idx = idx_v[pl.ds(i * L, L)]
            plsc.addupdate_scatter(acc_v, [idx], ones)
            return carry
        lax.fori_loop(0, ET // L, _count, 0)

        pltpu.sync_copy(acc_v, sacc.at[s])
        plsc.subcore_barrier()
        pltpu.sync_copy(sacc.at[:, pl.ds(s * (NP // NS), NP // NS)], slab_v)
        for j in range(40):
            v = slab_v[0, pl.ds(j * L, L)]
            for r in range(1, NS):
                v = v + slab_v[r, pl.ds(j * L, L)]
            i0 = plsc.bitcast(v, jnp.int32)
            y = plsc.bitcast(jnp.int32(0x5F3759DF) - (i0 >> 1), jnp.float32)
            for _ in range(3):
                y = y * (1.5 - 0.5 * v * y * y)
            dv_v[j] = jnp.where(v > 0.5, y, 0.0)
        pltpu.sync_copy(dv_v, dinv_hbm.at[pl.ds(s * 40, 40)])


def _make_p_kernel(C2):
    OR = NP // NS

    @functools.partial(
        pl.kernel,
        out_type=(jax.ShapeDtypeStruct((NP, C2), jnp.float32),
                  jax.ShapeDtypeStruct((NP, C2), jnp.float32)),
        mesh=_mesh(),
        scratch_types=[
            pltpu.VMEM((GB, KC), jnp.int32),
            pltpu.VMEM((GB, KC), jnp.int32),
            pltpu.VMEM((NB3, KC, C2), jnp.float32),
            pltpu.VMEM_SHARED((NP, C2), jnp.float32),
            pltpu.SemaphoreType.DMA,
            pltpu.SemaphoreType.DMA,
        ],
        compiler_params=_sc_params(),
        cost_estimate=pl.CostEstimate(
            flops=0, transcendentals=0, bytes_accessed=360_000_000),
    )
    def _p(v0, v1, src4, dst4, y0, y1, sidx, didx, rbuf, sacc, sem_g, sem_s):
        c = lax.axis_index("c")
        s = lax.axis_index("s")
        cl = C2 // L

        def body(v, y):
            zv = jnp.zeros((L,), jnp.float32)

            def _zr(i, carry):
                rbuf[0, i // cl, pl.ds((i % cl) * L, L)] = zv
                return carry
            lax.fori_loop(0, KC * cl, _zr, 0)
            for k in range(OR // KC):
                pltpu.sync_copy(rbuf.at[0],
                                sacc.at[pl.ds(s * OR + k * KC, KC)])
            plsc.subcore_barrier()

            for g in range(NG):
                pltpu.sync_copy(src4.at[s, g], sidx)
                pltpu.sync_copy(dst4.at[s, g], didx)
                pltpu.async_copy(v.at[sidx.at[0]], rbuf.at[0], sem_g)
                pltpu.async_copy(v.at[sidx.at[1]], rbuf.at[1], sem_g)
                pltpu.async_copy(v.at[sidx.at[2]], rbuf.at[2], sem_g)

                def chunk(k, carry):
                    b = k % NB3
                    pltpu.make_async_copy(
                        v.at[sidx.at[k]], rbuf.at[b], sem_g).wait()

                    @pl.when(k >= 1)
                    def _():
                        pltpu.make_async_copy(
                            rbuf.at[(k - 1) % NB3],
                            sacc.at[didx.at[k - 1]], sem_s).wait()

                    @pl.when(k + 3 < GB)
                    def _():
                        pltpu.async_copy(v.at[sidx.at[k + 3]],
                                         rbuf.at[(k + 3) % NB3], sem_g)
                    pltpu.async_copy(rbuf.at[b], sacc.at[didx.at[k]],
                                     sem_s, add=True)
                    return carry
                lax.fori_loop(0, GB, chunk, 0)
                pltpu.make_async_copy(
                    rbuf.at[(GB - 1) % NB3],
                    sacc.at[didx.at[GB - 1]], sem_s).wait()
            plsc.subcore_barrier()

            pltpu.sync_copy(sacc.at[pl.ds(s * OR, OR)], y.at[pl.ds(s * OR, OR)])

        @pl.when(c == 0)
        def _():
            body(v0, y0)

        @pl.when(c == 1)
        def _():
            body(v1, y1)

    return _p


PC = 128
_p_kernel_inst = []


def _p_apply(va, vb, src3, dst3):
    if not _p_kernel_inst:
        _p_kernel_inst.append(_make_p_kernel(PC))
    return _p_kernel_inst[0](va, vb, src3, dst3)


def _make_p2_kernel():
    C2 = PC
    OR = NP // NS
    NG2 = NG // NC

    @functools.partial(
        pl.kernel,
        out_type=(jax.ShapeDtypeStruct((NP, C2), jnp.float32),
                  jax.ShapeDtypeStruct((NP, C2), jnp.float32)),
        mesh=_mesh(),
        scratch_types=[
            pltpu.VMEM((GB, KC), jnp.int32),
            pltpu.VMEM((GB, KC), jnp.int32),
            pltpu.VMEM((NB3, KC, C2), jnp.float32),
            pltpu.VMEM_SHARED((NP, C2), jnp.float32),
            pltpu.SemaphoreType.DMA,
            pltpu.SemaphoreType.DMA,
        ],
        compiler_params=_sc_params(),
        cost_estimate=pl.CostEstimate(
            flops=0, transcendentals=0, bytes_accessed=200_000_000),
    )
    def _p2(v, src5, dst5, y0, y1, sidx, didx, rbuf, sacc, sem_g, sem_s):
        c = lax.axis_index("c")
        s = lax.axis_index("s")
        cl = C2 // L

        def body(y):
            zv = jnp.zeros((L,), jnp.float32)

            def _zr(i, carry):
                rbuf[0, i // cl, pl.ds((i % cl) * L, L)] = zv
                return carry
            lax.fori_loop(0, KC * cl, _zr, 0)
            for k in range(OR // KC):
                pltpu.sync_copy(rbuf.at[0],
                                sacc.at[pl.ds(s * OR + k * KC, KC)])
            plsc.subcore_barrier()

            for g in range(NG2):
                pltpu.sync_copy(src5.at[c, s, g], sidx)
                pltpu.sync_copy(dst5.at[c, s, g], didx)
                pltpu.async_copy(v.at[sidx.at[0]], rbuf.at[0], sem_g)
                pltpu.async_copy(v.at[sidx.at[1]], rbuf.at[1], sem_g)
                pltpu.async_copy(v.at[sidx.at[2]], rbuf.at[2], sem_g)

                def chunk(k, carry):
                    b = k % NB3
                    pltpu.make_async_copy(
                        v.at[sidx.at[k]], rbuf.at[b], sem_g).wait()

                    @pl.when(k >= 1)
                    def _():
                        pltpu.make_async_copy(
                            rbuf.at[(k - 1) % NB3],
                            sacc.at[didx.at[k - 1]], sem_s).wait()

                    @pl.when(k + 3 < GB)
                    def _():
                        pltpu.async_copy(v.at[sidx.at[k + 3]],
                                         rbuf.at[(k + 3) % NB3], sem_g)
                    pltpu.async_copy(rbuf.at[b], sacc.at[didx.at[k]],
                                     sem_s, add=True)
                    return carry
                lax.fori_loop(0, GB, chunk, 0)
                pltpu.make_async_copy(
                    rbuf.at[(GB - 1) % NB3],
                    sacc.at[didx.at[GB - 1]], sem_s).wait()
            plsc.subcore_barrier()
            pltpu.sync_copy(sacc.at[pl.ds(s * OR, OR)], y.at[pl.ds(s * OR, OR)])

        @pl.when(c == 0)
        def _():
            body(y0)

        @pl.when(c == 1)
        def _():
            body(y1)

    return _p2


_p2_kernel_inst = []


def _p2_apply(v, src5, dst5):
    if not _p2_kernel_inst:
        _p2_kernel_inst.append(_make_p2_kernel())
    return _p2_kernel_inst[0](v, src5, dst5)


def _lead(x, dinv_col, W, b):
    C = x.shape[1]
    C2 = C // 2
    Co = W.shape[2]

    def body(x_ref, d_ref, w_ref, b_ref, ua_ref, ub_ref, acc_ref):
        xv = x_ref[...]
        u = xv * d_ref[...]
        ua_ref[...] = u[:, :C2]
        ub_ref[...] = u[:, C2:]
        w0 = w_ref[0] - w_ref[2]
        acc_ref[...] = (jnp.dot(xv, w0, preferred_element_type=jnp.float32)
                        + b_ref[...])

    return pl.pallas_call(
        body,
        grid=(N // BR,),
        in_specs=[
            pl.BlockSpec((BR, C), lambda i: (i, 0)),
            pl.BlockSpec((BR, 1), lambda i: (i, 0)),
            pl.BlockSpec((3, C, Co), lambda i: (0, 0, 0)),
            pl.BlockSpec((1, Co), lambda i: (0, 0)),
        ],
        out_specs=[
            pl.BlockSpec((BR, C2), lambda i: (i, 0)),
            pl.BlockSpec((BR, C2), lambda i: (i, 0)),
            pl.BlockSpec((BR, Co), lambda i: (i, 0)),
        ],
        out_shape=[
            jax.ShapeDtypeStruct((N, C2), jnp.float32),
            jax.ShapeDtypeStruct((N, C2), jnp.float32),
            jax.ShapeDtypeStruct((N, Co), jnp.float32),
        ],
    )(x, dinv_col, W, b)


def _scale1(sa, sb, dinv_col, C2):
    def body(sa_ref, sb_ref, d_ref, ua_ref, ub_ref):
        d = d_ref[...]
        f = -(d * d)
        ua_ref[...] = sa_ref[...] * f
        ub_ref[...] = sb_ref[...] * f

    return pl.pallas_call(
        body,
        grid=(N // BR,),
        in_specs=[
            pl.BlockSpec((BR, PC), lambda i: (i, 0)),
            pl.BlockSpec((BR, PC), lambda i: (i, 0)),
            pl.BlockSpec((BR, 1), lambda i: (i, 0)),
        ],
        out_specs=[
            pl.BlockSpec((BR, PC), lambda i: (i, 0)),
            pl.BlockSpec((BR, PC), lambda i: (i, 0)),
        ],
        out_shape=[
            jax.ShapeDtypeStruct((N, PC), jnp.float32),
            jax.ShapeDtypeStruct((N, PC), jnp.float32),
        ],
    )(sa, sb, dinv_col)


def _scale1_sum(sa, sb, dinv_col):
    def body(sa_ref, sb_ref, d_ref, u_ref):
        d = d_ref[...]
        u_ref[...] = (sa_ref[...] + sb_ref[...]) * (-(d * d))

    return pl.pallas_call(
        body,
        grid=(N // BR,),
        in_specs=[
            pl.BlockSpec((BR, PC), lambda i: (i, 0)),
            pl.BlockSpec((BR, PC), lambda i: (i, 0)),
            pl.BlockSpec((BR, 1), lambda i: (i, 0)),
        ],
        out_specs=pl.BlockSpec((BR, PC), lambda i: (i, 0)),
        out_shape=jax.ShapeDtypeStruct((N, PC), jnp.float32),
    )(sa, sb, dinv_col)


def _post_pre(acc0, s1a, s1b, s2a, s2b, dinv_col, W, Wn, bn, s_mode, u_out):
    Ci = W.shape[1]
    Co = W.shape[2]
    Con = Wn.shape[2]
    C2o = Co // 2

    def body(a_ref, s1a_ref, s1b_ref, s2a_ref, s2b_ref, d_ref, w_ref,
             wn_ref, bn_ref, an_ref, *u_refs):
        d = d_ref[...]
        if s_mode == "halves":
            s1 = jnp.concatenate([s1a_ref[...], s1b_ref[...]], axis=1) * d
            s2 = jnp.concatenate([s2a_ref[...], s2b_ref[...]], axis=1) * d
        else:
            s1 = (s1a_ref[...] + s1b_ref[...]) * d
            s2 = (s2a_ref[...] + s2b_ref[...]) * d
        acc = a_ref[...] - jnp.dot(s1, w_ref[1],
                                   preferred_element_type=jnp.float32)
        acc -= 2.0 * jnp.dot(s2, w_ref[2], preferred_element_type=jnp.float32)
        h_out = jnp.maximum(acc, 0.0)
        wn0 = wn_ref[0] - wn_ref[2]
        an_ref[...] = (jnp.dot(h_out, wn0, preferred_element_type=jnp.float32)
                       + bn_ref[...])
        if u_out == "full":
            u_refs[0][...] = h_out * d
        elif u_out == "halves":
            u = h_out * d
            u_refs[0][...] = u[:, :C2o]
            u_refs[1][...] = u[:, C2o:]

    out_shape = [jax.ShapeDtypeStruct((N, Con), jnp.float32)]
    out_specs = [pl.BlockSpec((BR, Con), lambda i: (i, 0))]
    if u_out == "full":
        out_shape += [jax.ShapeDtypeStruct((N, Co), jnp.float32)]
        out_specs += [pl.BlockSpec((BR, Co), lambda i: (i, 0))]
    elif u_out == "halves":
        out_shape += [jax.ShapeDtypeStruct((N, C2o), jnp.float32)] * 2
        out_specs += [pl.BlockSpec((BR, C2o), lambda i: (i, 0))] * 2

    return pl.pallas_call(
        body,
        grid=(N // BR,),
        in_specs=[
            pl.BlockSpec((BR, Co), lambda i: (i, 0)),
            pl.BlockSpec((BR, PC), lambda i: (i, 0)),
            pl.BlockSpec((BR, PC), lambda i: (i, 0)),
            pl.BlockSpec((BR, PC), lambda i: (i, 0)),
            pl.BlockSpec((BR, PC), lambda i: (i, 0)),
            pl.BlockSpec((BR, 1), lambda i: (i, 0)),
            pl.BlockSpec((3, Ci, Co), lambda i: (0, 0, 0)),
            pl.BlockSpec((3, Co, Con), lambda i: (0, 0, 0)),
            pl.BlockSpec((1, Con), lambda i: (0, 0)),
        ],
        out_specs=out_specs,
        out_shape=out_shape,
    )(acc0, s1a, s1b, s2a, s2b, dinv_col, W, Wn, bn)


def _cheb_post_pool(acc0, s1a, s1b, s2a, s2b, dinv_col, W, batch3, Wfc,
                    bfc_row):
    Ci = W.shape[1]
    Co = W.shape[2]
    NB = N // BR

    def body(a_ref, s1a_ref, s1b_ref, s2a_ref, s2b_ref, d_ref, w_ref,
             batch_ref, wfc_ref, bfc_ref, out_ref, sums, cnt):
        i = pl.program_id(0)

        @pl.when(i == 0)
        def _():
            sums[...] = jnp.zeros_like(sums)
            cnt[...] = jnp.zeros_like(cnt)

        d = d_ref[...]
        s1 = jnp.concatenate([s1a_ref[...], s1b_ref[...]], axis=1) * d
        s2 = jnp.concatenate([s2a_ref[...], s2b_ref[...]], axis=1) * d
        acc = a_ref[...] - jnp.dot(s1, w_ref[1],
                                   preferred_element_type=jnp.float32)
        acc -= 2.0 * jnp.dot(s2, w_ref[2], preferred_element_type=jnp.float32)
        h3 = jnp.maximum(acc, 0.0)

        mt = (lax.broadcasted_iota(jnp.int32, (G, BR), 0)
              == batch_ref[...][0]).astype(jnp.float32)
        sums[...] += jnp.dot(mt, h3, preferred_element_type=jnp.float32)
        cnt[...] += jnp.sum(mt, axis=1, keepdims=True)

        @pl.when(i == NB - 1)
        def _():
            pooled = sums[...] / jnp.maximum(cnt[...], 1.0)
            out_ref[...] = (jnp.dot(pooled, wfc_ref[...],
                                    preferred_element_type=jnp.float32)
                            + bfc_ref[...])

    return pl.pallas_call(
        body,
        grid=(NB,),
        in_specs=[
            pl.BlockSpec((BR, Co), lambda i: (i, 0)),
            pl.BlockSpec((BR, PC), lambda i: (i, 0)),
            pl.BlockSpec((BR, PC), lambda i: (i, 0)),
            pl.BlockSpec((BR, PC), lambda i: (i, 0)),
            pl.BlockSpec((BR, PC), lambda i: (i, 0)),
            pl.BlockSpec((BR, 1), lambda i: (i, 0)),
            pl.BlockSpec((3, Ci, Co), lambda i: (0, 0, 0)),
            pl.BlockSpec((1, 1, BR), lambda i: (i, 0, 0)),
            pl.BlockSpec(Wfc.shape, lambda i: (0, 0)),
            pl.BlockSpec((1, 128), lambda i: (0, 0)),
        ],
        out_specs=pl.BlockSpec((G, 128), lambda i: (0, 0)),
        out_shape=jax.ShapeDtypeStruct((G, 128), jnp.float32),
        scratch_shapes=[
            pltpu.VMEM((G, Co), jnp.float32),
            pltpu.VMEM((G, 1), jnp.float32),
        ],
    )(acc0, s1a, s1b, s2a, s2b, dinv_col, W, batch3, Wfc, bfc_row)


def kernel(x, edge_index, batch, W1, b1, W2, b2, W3, b3, Wfc, bfc):
    src = edge_index[0]
    dst = edge_index[1]
    src_deg = src.reshape(NS, ET)
    pad = EP - E
    src_p = jnp.concatenate(
        [src, (jnp.arange(pad, dtype=jnp.int32) * 13) % N])
    dst_p = jnp.concatenate([dst, jnp.full((pad,), NP - 1, jnp.int32)])
    src3 = src_p.reshape(NS, NG, GB, KC)
    dst3 = dst_p.reshape(NS, NG, GB, KC)
    src5 = src_p.reshape(NC, NS, NG // NC, GB, KC)
    dst5 = dst_p.reshape(NC, NS, NG // NC, GB, KC)

    dinv2d = _deg_dinv_kernel(src_deg)
    dinv_col = dinv2d.reshape(NP)[:N].reshape(N, 1)

    ua, ub, pre1 = _lead(x, dinv_col, W1, b1.reshape(1, -1))
    s1a, s1b = _p_apply(ua, ub, src3, dst3)
    u1a, u1b = _scale1(s1a, s1b, dinv_col, PC)
    s2a, s2b = _p_apply(u1a, u1b, src3, dst3)
    pre2, uf = _post_pre(pre1, s1a, s1b, s2a, s2b, dinv_col, W1, W2,
                         b2.reshape(1, -1), s_mode="halves", u_out="full")

    t1a, t1b = _p2_apply(uf, src5, dst5)
    u2f = _scale1_sum(t1a, t1b, dinv_col)
    t2a, t2b = _p2_apply(u2f, src5, dst5)
    pre3, ua, ub = _post_pre(pre2, t1a, t1b, t2a, t2b, dinv_col, W2, W3,
                             b3.reshape(1, -1), s_mode="partials",
                             u_out="halves")

    s1a, s1b = _p_apply(ua, ub, src3, dst3)
    u1a, u1b = _scale1(s1a, s1b, dinv_col, PC)
    s2a, s2b = _p_apply(u1a, u1b, src3, dst3)
    return _cheb_post_pool(pre3, s1a, s1b, s2a, s2b, dinv_col, W3,
                           batch.reshape(N // BR, 1, BR), Wfc,
                           bfc.reshape(1, -1))


# speedup vs baseline: 1.2490x; 1.0246x over previous
import functools

import jax
import jax.numpy as jnp
from jax import lax
from jax.experimental import pallas as pl
from jax.experimental.pallas import tpu as pltpu
from jax.experimental.pallas import tpu_sc as plsc

N = 10000
E = 160000
G = 128
NP = 10240
NC = 2
NS = 16
L = 16
BR = 1000

ET = E // NS
KC = 64
GB = 20
NG = 8
NB3 = 4
EP = NS * NG * GB * KC

_MESH = dict(core_axis_name="c", subcore_axis_name="s",
             num_cores=NC, num_subcores=NS)


def _mesh():
    return plsc.VectorSubcoreMesh(**_MESH)


def _sc_params():
    return pltpu.CompilerParams(needs_layout_passes=False)


@functools.partial(
    pl.kernel,
    out_type=jax.ShapeDtypeStruct((NP // L, L), jnp.float32),
    mesh=_mesh(),
    scratch_types=[
        pltpu.VMEM((ET,), jnp.int32),
        pltpu.VMEM((NP,), jnp.float32),
        pltpu.VMEM((NS, NP // NS), jnp.float32),
        pltpu.VMEM((40, L), jnp.float32),
        pltpu.VMEM_SHARED((NS, NP), jnp.float32),
    ],
    compiler_params=_sc_params(),
)
def _deg_dinv_kernel(src_hbm, dinv_hbm, idx_v, acc_v, slab_v, dv_v, sacc):
    c = lax.axis_index("c")
    s = lax.axis_index("s")

    @pl.when(c == 0)
    def _():
        zv = jnp.zeros((L,), jnp.float32)

        def _zero(i, carry):
            acc_v[pl.ds(i * L, L)] = zv
            return carry
        lax.fori_loop(0, NP // L, _zero, 0)

        pltpu.sync_copy(src_hbm.at[s], idx_v)
        ones = jnp.full((L,), 1.0, jnp.float32)

        def _count(i, carry):
            idx = idx_v[pl.ds(i * L, L)]
            plsc.addupdate_scatter(acc_v, [idx], ones)
            return carry
        lax.fori_loop(0, ET // L, _count, 0)

        pltpu.sync_copy(acc_v, sacc.at[s])
        plsc.subcore_barrier()
        pltpu.sync_copy(sacc.at[:, pl.ds(s * (NP // NS), NP // NS)], slab_v)
        for j in range(40):
            v = slab_v[0, pl.ds(j * L, L)]
            for r in range(1, NS):
                v = v + slab_v[r, pl.ds(j * L, L)]
            i0 = plsc.bitcast(v, jnp.int32)
            y = plsc.bitcast(jnp.int32(0x5F3759DF) - (i0 >> 1), jnp.float32)
            for _ in range(3):
                y = y * (1.5 - 0.5 * v * y * y)
            dv_v[j] = jnp.where(v > 0.5, y, 0.0)
        pltpu.sync_copy(dv_v, dinv_hbm.at[pl.ds(s * 40, 40)])


def _make_p_kernel(C2):
    OR = NP // NS

    @functools.partial(
        pl.kernel,
        out_type=(jax.ShapeDtypeStruct((NP, C2), jnp.float32),
                  jax.ShapeDtypeStruct((NP, C2), jnp.float32)),
        mesh=_mesh(),
        scratch_types=[
            pltpu.VMEM((GB, KC), jnp.int32),
            pltpu.VMEM((GB, KC), jnp.int32),
            pltpu.VMEM((NB3, KC, C2), jnp.float32),
            pltpu.VMEM_SHARED((NP, C2), jnp.float32),
            pltpu.SemaphoreType.DMA,
            pltpu.SemaphoreType.DMA,
        ],
        compiler_params=_sc_params(),
        cost_estimate=pl.CostEstimate(
            flops=0, transcendentals=0, bytes_accessed=360_000_000),
    )
    def _p(v0, v1, src4, dst4, y0, y1, sidx, didx, rbuf, sacc, sem_g, sem_s):
        c = lax.axis_index("c")
        s = lax.axis_index("s")
        cl = C2 // L

        def body(v, y):
            zv = jnp.zeros((L,), jnp.float32)

            def _zr(i, carry):
                rbuf[0, i // cl, pl.ds((i % cl) * L, L)] = zv
                return carry
            lax.fori_loop(0, KC * cl, _zr, 0)
            for k in range(OR // KC):
                pltpu.sync_copy(rbuf.at[0],
                                sacc.at[pl.ds(s * OR + k * KC, KC)])
            plsc.subcore_barrier()

            for g in range(NG):
                pltpu.sync_copy(src4.at[s, g], sidx)
                pltpu.sync_copy(dst4.at[s, g], didx)
                pltpu.async_copy(v.at[sidx.at[0]], rbuf.at[0], sem_g)
                pltpu.async_copy(v.at[sidx.at[1]], rbuf.at[1], sem_g)
                pltpu.async_copy(v.at[sidx.at[2]], rbuf.at[2], sem_g)

                def chunk(k, carry):
                    b = k % NB3
                    pltpu.make_async_copy(
                        v.at[sidx.at[k]], rbuf.at[b], sem_g).wait()

                    @pl.when(k >= 1)
                    def _():
                        pltpu.make_async_copy(
                            rbuf.at[(k - 1) % NB3],
                            sacc.at[didx.at[k - 1]], sem_s).wait()

                    @pl.when(k + 3 < GB)
                    def _():
                        pltpu.async_copy(v.at[sidx.at[k + 3]],
                                         rbuf.at[(k + 3) % NB3], sem_g)
                    pltpu.async_copy(rbuf.at[b], sacc.at[didx.at[k]],
                                     sem_s, add=True)
                    return carry
                lax.fori_loop(0, GB, chunk, 0)
                pltpu.make_async_copy(
                    rbuf.at[(GB - 1) % NB3],
                    sacc.at[didx.at[GB - 1]], sem_s).wait()
            plsc.subcore_barrier()

            pltpu.sync_copy(sacc.at[pl.ds(s * OR, OR)], y.at[pl.ds(s * OR, OR)])

        @pl.when(c == 0)
        def _():
            body(v0, y0)

        @pl.when(c == 1)
        def _():
            body(v1, y1)

    return _p


PC = 128
_p_kernel_inst = []


def _p_apply(va, vb, src3, dst3):
    if not _p_kernel_inst:
        _p_kernel_inst.append(_make_p_kernel(PC))
    return _p_kernel_inst[0](va, vb, src3, dst3)


def _make_p2_kernel():
    C2 = PC
    OR = NP // NS
    NG2 = NG // NC

    @functools.partial(
        pl.kernel,
        out_type=(jax.ShapeDtypeStruct((NP, C2), jnp.float32),
                  jax.ShapeDtypeStruct((NP, C2), jnp.float32)),
        mesh=_mesh(),
        scratch_types=[
            pltpu.VMEM((GB, KC), jnp.int32),
            pltpu.VMEM((GB, KC), jnp.int32),
            pltpu.VMEM((NB3, KC, C2), jnp.float32),
            pltpu.VMEM_SHARED((NP, C2), jnp.float32),
            pltpu.SemaphoreType.DMA,
            pltpu.SemaphoreType.DMA,
        ],
        compiler_params=_sc_params(),
        cost_estimate=pl.CostEstimate(
            flops=0, transcendentals=0, bytes_accessed=200_000_000),
    )
    def _p2(v, src5, dst5, y0, y1, sidx, didx, rbuf, sacc, sem_g, sem_s):
        c = lax.axis_index("c")
        s = lax.axis_index("s")
        cl = C2 // L

        def body(y):
            zv = jnp.zeros((L,), jnp.float32)

            def _zr(i, carry):
                rbuf[0, i // cl, pl.ds((i % cl) * L, L)] = zv
                return carry
            lax.fori_loop(0, KC * cl, _zr, 0)
            for k in range(OR // KC):
                pltpu.sync_copy(rbuf.at[0],
                                sacc.at[pl.ds(s * OR + k * KC, KC)])
            plsc.subcore_barrier()

            for g in range(NG2):
                pltpu.sync_copy(src5.at[c, s, g], sidx)
                pltpu.sync_copy(dst5.at[c, s, g], didx)
                pltpu.async_copy(v.at[sidx.at[0]], rbuf.at[0], sem_g)
                pltpu.async_copy(v.at[sidx.at[1]], rbuf.at[1], sem_g)
                pltpu.async_copy(v.at[sidx.at[2]], rbuf.at[2], sem_g)

                def chunk(k, carry):
                    b = k % NB3
                    pltpu.make_async_copy(
                        v.at[sidx.at[k]], rbuf.at[b], sem_g).wait()

                    @pl.when(k >= 1)
                    def _():
                        pltpu.make_async_copy(
                            rbuf.at[(k - 1) % NB3],
                            sacc.at[didx.at[k - 1]], sem_s).wait()

                    @pl.when(k + 3 < GB)
                    def _():
                        pltpu.async_copy(v.at[sidx.at[k + 3]],
                                         rbuf.at[(k + 3) % NB3], sem_g)
                    pltpu.async_copy(rbuf.at[b], sacc.at[didx.at[k]],
                                     sem_s, add=True)
                    return carry
                lax.fori_loop(0, GB, chunk, 0)
                pltpu.make_async_copy(
                    rbuf.at[(GB - 1) % NB3],
                    sacc.at[didx.at[GB - 1]], sem_s).wait()
            plsc.subcore_barrier()
            pltpu.sync_copy(sacc.at[pl.ds(s * OR, OR)], y.at[pl.ds(s * OR, OR)])

        @pl.when(c == 0)
        def _():
            body(y0)

        @pl.when(c == 1)
        def _():
            body(y1)

    return _p2


_p2_kernel_inst = []


def _p2_apply(v, src5, dst5):
    if not _p2_kernel_inst:
        _p2_kernel_inst.append(_make_p2_kernel())
    return _p2_kernel_inst[0](v, src5, dst5)


def _lead(x, dinv_col, W, b):
    C = x.shape[1]
    C2 = C // 2
    Co = W.shape[2]

    def body(x_ref, d_ref, w_ref, b_ref, ua_ref, ub_ref, acc_ref):
        xv = x_ref[...]
        u = xv * d_ref[...]
        ua_ref[...] = u[:, :C2]
        ub_ref[...] = u[:, C2:]
        w0 = w_ref[0] - w_ref[2]
        acc_ref[...] = (jnp.dot(xv, w0, preferred_element_type=jnp.float32)
                        + b_ref[...])

    return pl.pallas_call(
        body,
        grid=(N // BR,),
        in_specs=[
            pl.BlockSpec((BR, C), lambda i: (i, 0)),
            pl.BlockSpec((BR, 1), lambda i: (i, 0)),
            pl.BlockSpec((3, C, Co), lambda i: (0, 0, 0)),
            pl.BlockSpec((1, Co), lambda i: (0, 0)),
        ],
        out_specs=[
            pl.BlockSpec((BR, C2), lambda i: (i, 0)),
            pl.BlockSpec((BR, C2), lambda i: (i, 0)),
            pl.BlockSpec((BR, Co), lambda i: (i, 0)),
        ],
        out_shape=[
            jax.ShapeDtypeStruct((N, C2), jnp.float32),
            jax.ShapeDtypeStruct((N, C2), jnp.float32),
            jax.ShapeDtypeStruct((N, Co), jnp.float32),
        ],
    )(x, dinv_col, W, b)


def _scale1(sa, sb, dinv_col, C2):
    def body(sa_ref, sb_ref, d_ref, ua_ref, ub_ref):
        d = d_ref[...]
        f = -(d * d)
        ua_ref[...] = sa_ref[...] * f
        ub_ref[...] = sb_ref[...] * f

    return pl.pallas_call(
        body,
        grid=(N // BR,),
        in_specs=[
            pl.BlockSpec((BR, PC), lambda i: (i, 0)),
            pl.BlockSpec((BR, PC), lambda i: (i, 0)),
            pl.BlockSpec((BR, 1), lambda i: (i, 0)),
        ],
        out_specs=[
            pl.BlockSpec((BR, PC), lambda i: (i, 0)),
            pl.BlockSpec((BR, PC), lambda i: (i, 0)),
        ],
        out_shape=[
            jax.ShapeDtypeStruct((N, PC), jnp.float32),
            jax.ShapeDtypeStruct((N, PC), jnp.float32),
        ],
    )(sa, sb, dinv_col)


def _scale1_sum(sa, sb, dinv_col):
    def body(sa_ref, sb_ref, d_ref, u_ref):
        d = d_ref[...]
        u_ref[...] = (sa_ref[...] + sb_ref[...]) * (-(d * d))

    return pl.pallas_call(
        body,
        grid=(N // BR,),
        in_specs=[
            pl.BlockSpec((BR, PC), lambda i: (i, 0)),
            pl.BlockSpec((BR, PC), lambda i: (i, 0)),
            pl.BlockSpec((BR, 1), lambda i: (i, 0)),
        ],
        out_specs=pl.BlockSpec((BR, PC), lambda i: (i, 0)),
        out_shape=jax.ShapeDtypeStruct((N, PC), jnp.float32),
    )(sa, sb, dinv_col)


def _post_pre(acc0, s1a, s1b, s2a, s2b, dinv_col, W, Wn, bn, s_mode, u_out):
    Ci = W.shape[1]
    Co = W.shape[2]
    Con = Wn.shape[2]
    C2o = Co // 2

    def body(a_ref, s1a_ref, s1b_ref, s2a_ref, s2b_ref, d_ref, w_ref,
             wn_ref, bn_ref, an_ref, *u_refs):
        d = d_ref[...]
        if s_mode == "halves":
            s1 = jnp.concatenate([s1a_ref[...], s1b_ref[...]], axis=1) * d
            s2 = jnp.concatenate([s2a_ref[...], s2b_ref[...]], axis=1) * d
        else:
            s1 = (s1a_ref[...] + s1b_ref[...]) * d
            s2 = (s2a_ref[...] + s2b_ref[...]) * d
        acc = a_ref[...] - jnp.dot(s1, w_ref[1],
                                   preferred_element_type=jnp.float32)
        acc -= 2.0 * jnp.dot(s2, w_ref[2], preferred_element_type=jnp.float32)
        h_out = jnp.maximum(acc, 0.0)
        wn0 = wn_ref[0] - wn_ref[2]
        an_ref[...] = (jnp.dot(h_out, wn0, preferred_element_type=jnp.float32)
                       + bn_ref[...])
        if u_out == "full":
            u_refs[0][...] = h_out * d
        elif u_out == "halves":
            u = h_out * d
            u_refs[0][...] = u[:, :C2o]
            u_refs[1][...] = u[:, C2o:]

    out_shape = [jax.ShapeDtypeStruct((N, Con), jnp.float32)]
    out_specs = [pl.BlockSpec((BR, Con), lambda i: (i, 0))]
    if u_out == "full":
        out_shape += [jax.ShapeDtypeStruct((N, Co), jnp.float32)]
        out_specs += [pl.BlockSpec((BR, Co), lambda i: (i, 0))]
    elif u_out == "halves":
        out_shape += [jax.ShapeDtypeStruct((N, C2o), jnp.float32)] * 2
        out_specs += [pl.BlockSpec((BR, C2o), lambda i: (i, 0))] * 2

    return pl.pallas_call(
        body,
        grid=(N // BR,),
        in_specs=[
            pl.BlockSpec((BR, Co), lambda i: (i, 0)),
            pl.BlockSpec((BR, PC), lambda i: (i, 0)),
            pl.BlockSpec((BR, PC), lambda i: (i, 0)),
            pl.BlockSpec((BR, PC), lambda i: (i, 0)),
            pl.BlockSpec((BR, PC), lambda i: (i, 0)),
            pl.BlockSpec((BR, 1), lambda i: (i, 0)),
            pl.BlockSpec((3, Ci, Co), lambda i: (0, 0, 0)),
            pl.BlockSpec((3, Co, Con), lambda i: (0, 0, 0)),
            pl.BlockSpec((1, Con), lambda i: (0, 0)),
        ],
        out_specs=out_specs,
        out_shape=out_shape,
    )(acc0, s1a, s1b, s2a, s2b, dinv_col, W, Wn, bn)


def _cheb_post_pool(acc0, s1a, s1b, s2a, s2b, dinv_col, W, batch3, Wfc,
                    bfc_row):
    Ci = W.shape[1]
    Co = W.shape[2]
    NB = N // BR

    def body(a_ref, s1a_ref, s1b_ref, s2a_ref, s2b_ref, d_ref, w_ref,
             batch_ref, wfc_ref, bfc_ref, out_ref, sums, cnt):
        i = pl.program_id(0)

        @pl.when(i == 0)
        def _():
            sums[...] = jnp.zeros_like(sums)
            cnt[...] = jnp.zeros_like(cnt)

        d = d_ref[...]
        s1 = jnp.concatenate([s1a_ref[...], s1b_ref[...]], axis=1) * d
        s2 = jnp.concatenate([s2a_ref[...], s2b_ref[...]], axis=1) * d
        acc = a_ref[...] - jnp.dot(s1, w_ref[1],
                                   preferred_element_type=jnp.float32)
        acc -= 2.0 * jnp.dot(s2, w_ref[2], preferred_element_type=jnp.float32)
        h3 = jnp.maximum(acc, 0.0)

        mt = (lax.broadcasted_iota(jnp.int32, (G, BR), 0)
              == batch_ref[...][0]).astype(jnp.float32)
        sums[...] += jnp.dot(mt, h3, preferred_element_type=jnp.float32)
        cnt[...] += jnp.sum(mt, axis=1, keepdims=True)

        @pl.when(i == NB - 1)
        def _():
            pooled = sums[...] / jnp.maximum(cnt[...], 1.0)
            out_ref[...] = (jnp.dot(pooled, wfc_ref[...],
                                    preferred_element_type=jnp.float32)
                            + bfc_ref[...])

    return pl.pallas_call(
        body,
        grid=(NB,),
        in_specs=[
            pl.BlockSpec((BR, Co), lambda i: (i, 0)),
            pl.BlockSpec((BR, PC), lambda i: (i, 0)),
            pl.BlockSpec((BR, PC), lambda i: (i, 0)),
            pl.BlockSpec((BR, PC), lambda i: (i, 0)),
            pl.BlockSpec((BR, PC), lambda i: (i, 0)),
            pl.BlockSpec((BR, 1), lambda i: (i, 0)),
            pl.BlockSpec((3, Ci, Co), lambda i: (0, 0, 0)),
            pl.BlockSpec((1, 1, BR), lambda i: (i, 0, 0)),
            pl.BlockSpec(Wfc.shape, lambda i: (0, 0)),
            pl.BlockSpec((1, 128), lambda i: (0, 0)),
        ],
        out_specs=pl.BlockSpec((G, 128), lambda i: (0, 0)),
        out_shape=jax.ShapeDtypeStruct((G, 128), jnp.float32),
        scratch_shapes=[
            pltpu.VMEM((G, Co), jnp.float32),
            pltpu.VMEM((G, 1), jnp.float32),
        ],
    )(acc0, s1a, s1b, s2a, s2b, dinv_col, W, batch3, Wfc, bfc_row)


def kernel(x, edge_index, batch, W1, b1, W2, b2, W3, b3, Wfc, bfc):
    src = edge_index[0]
    dst = edge_index[1]
    src_deg = src.reshape(NS, ET)
    pad = EP - E
    src_p = jnp.concatenate(
        [src, (jnp.arange(pad, dtype=jnp.int32) * 13) % N])
    dst_p = jnp.concatenate([dst, jnp.full((pad,), NP - 1, jnp.int32)])
    src3 = src_p.reshape(NS, NG, GB, KC)
    dst3 = dst_p.reshape(NS, NG, GB, KC)
    src5 = src_p.reshape(NC, NS, NG // NC, GB, KC)
    dst5 = dst_p.reshape(NC, NS, NG // NC, GB, KC)

    dinv2d = _deg_dinv_kernel(src_deg)
    dinv_col = dinv2d.reshape(NP)[:N].reshape(N, 1)

    ua, ub, pre1 = _lead(x, dinv_col, W1, b1.reshape(1, -1))
    s1a, s1b = _p_apply(ua, ub, src3, dst3)
    u1a, u1b = _scale1(s1a, s1b, dinv_col, PC)
    s2a, s2b = _p_apply(u1a, u1b, src3, dst3)
    pre2, uf = _post_pre(pre1, s1a, s1b, s2a, s2b, dinv_col, W1, W2,
                         b2.reshape(1, -1), s_mode="halves", u_out="full")

    t1a, t1b = _p2_apply(uf, src5, dst5)
    u2f = _scale1_sum(t1a, t1b, dinv_col)
    t2a, t2b = _p2_apply(u2f, src5, dst5)
    pre3, ua, ub = _post_pre(pre2, t1a, t1b, t2a, t2b, dinv_col, W2, W3,
                             b3.reshape(1, -1), s_mode="partials",
                             u_out="halves")

    s1a, s1b = _p_apply(ua, ub, src3, dst3)
    u1a, u1b = _scale1(s1a, s1b, dinv_col, PC)
    s2a, s2b = _p_apply(u1a, u1b, src3, dst3)
    return _cheb_post_pool(pre3, s1a, s1b, s2a, s2b, dinv_col, W3,
                           batch.reshape(N // BR, 1, BR), Wfc,
                           bfc.reshape(1, -1))
